# Initial kernel scaffold; baseline (speedup 1.0000x reference)
import jax
import jax.numpy as jnp
from jax.experimental import pallas as pl


def kernel(logits, label):
    raise NotImplementedError("write your pallas kernel here")


# speedup vs baseline: 53.7012x; 53.7012x over previous
import functools

import jax
import jax.numpy as jnp
from jax import lax
from jax.experimental import pallas as pl
from jax.experimental.pallas import tpu as pltpu
from jax.experimental.pallas import tpu_sc as plsc

NCLS = 19
IGN = 255
EPS = 0.001

B = 4
HW = 512 * 512
NPIX = B * HW
NW = 32
P_W = NPIX // NW
PARTS = HW // P_W
CHUNK = 2048
NCH = P_W // CHUNK
VEC = 16
NV = CHUNK // VEC


def _issue(lg, lab, b19, g0, q0, xb, lb, sem):
    for c in range(NCLS):
        pltpu.make_async_copy(
            lg.at[pl.ds((b19 + c) * HW + q0, CHUNK)],
            xb.at[pl.ds(c * CHUNK, CHUNK)], sem
        ).start()
    pltpu.make_async_copy(lab.at[pl.ds(g0, CHUNK)], lb, sem).start()


def _wait(lg, lab, xb, lb, sem):
    for c in range(NCLS):
        pltpu.make_async_copy(
            lg.at[pl.ds(0, CHUNK)], xb.at[pl.ds(c * CHUNK, CHUNK)], sem
        ).wait()
    pltpu.make_async_copy(lab.at[pl.ds(0, CHUNK)], lb, sem).wait()


def _chunk_compute(xb, lb, carry, iota16):
    def body(i, carry):
        ai, ap, ac = carry
        base = i * VEC
        labv = lb[pl.ds(base, VEC)]
        valid = labv != IGN
        slab = jnp.where(valid, labv, 0)
        nz = jnp.logical_and(valid, labv != 0)
        e0 = jnp.exp(xb[pl.ds(base, VEC)])
        s = e0
        for c in range(1, NCLS):
            s = s + jnp.exp(xb[pl.ds(c * CHUNK + base, VEC)])
        xl = plsc.load_gather(xb, [slab * CHUNK + (iota16 + base)])
        el = jnp.exp(xl)
        r = 1.0 / s
        nzf = jnp.where(nz, 1.0, 0.0)
        ai = ai + nzf * el * r
        ap = ap + jnp.where(valid, (s - e0) * r, 0.0)
        ac = ac + nzf
        return ai, ap, ac

    return lax.fori_loop(0, NV, body, carry)


def _dice_body(lg, lab, out, xb0, xb1, lb0, lb1, ob, sem0, sem1):
    wid = lax.axis_index("s") * 2 + lax.axis_index("c")
    b = wid // PARTS
    off = (wid % PARTS) * P_W
    b19 = b * NCLS
    g0 = b * HW + off
    iota16 = lax.iota(jnp.int32, VEC)

    _issue(lg, lab, b19, g0, off, xb0, lb0, sem0)

    def outer(kk, carry):
        qa = off + (2 * kk) * CHUNK
        _issue(lg, lab, b19, g0 + qa - off + CHUNK, qa + CHUNK, xb1, lb1, sem1)
        _wait(lg, lab, xb0, lb0, sem0)
        carry = _chunk_compute(xb0, lb0, carry, iota16)

        @pl.when(kk < NCH // 2 - 1)
        def _():
            _issue(lg, lab, b19, g0 + qa - off + 2 * CHUNK, qa + 2 * CHUNK,
                   xb0, lb0, sem0)

        _wait(lg, lab, xb1, lb1, sem1)
        carry = _chunk_compute(xb1, lb1, carry, iota16)
        return carry

    z = jnp.zeros((VEC,), jnp.float32)
    ai, ap, ac = lax.fori_loop(0, NCH // 2, outer, (z, z, z))

    ob[pl.ds(0, VEC)] = ai
    ob[pl.ds(VEC, VEC)] = ap
    ob[pl.ds(2 * VEC, VEC)] = ac
    pltpu.sync_copy(ob, out.at[pl.ds(wid * 3 * VEC, 3 * VEC)])


@functools.lru_cache(maxsize=1)
def _dice_sc():
    return pl.kernel(
        _dice_body,
        out_type=jax.ShapeDtypeStruct((NW * 3 * VEC,), jnp.float32),
        mesh=plsc.VectorSubcoreMesh(core_axis_name="c", subcore_axis_name="s"),
        compiler_params=pltpu.CompilerParams(needs_layout_passes=False),
        scratch_types=[
            pltpu.VMEM((NCLS * CHUNK,), jnp.float32),
            pltpu.VMEM((NCLS * CHUNK,), jnp.float32),
            pltpu.VMEM((CHUNK,), jnp.int32),
            pltpu.VMEM((CHUNK,), jnp.int32),
            pltpu.VMEM((3 * VEC,), jnp.float32),
            pltpu.SemaphoreType.DMA,
            pltpu.SemaphoreType.DMA,
        ],
    )


def kernel(logits, label):
    lg = logits.reshape(B * NCLS * HW)
    lab = label.reshape(NPIX)
    parts = _dice_sc()(lg, lab).reshape(NW, 3 * VEC)
    si = jnp.sum(parts[:, 0:VEC])
    sp = jnp.sum(parts[:, VEC:2 * VEC])
    sc = jnp.sum(parts[:, 2 * VEC:3 * VEC])
    return 1.0 - (2.0 * si + EPS) / (sp + sc + EPS)
